# baseline (speedup 1.0000x reference)
import functools

import jax
import jax.numpy as jnp
from jax import lax
from jax.experimental import pallas as pl
from jax.experimental.pallas import tpu as pltpu
from jax.experimental.pallas import tpu_sc as plsc

N = 10000
E = 320000
D = 128
NC = 2
NS = 16
CWA = 125
ROWS_ALL_A = E // CWA
ROWS_A = ROWS_ALL_A // NS
CW = 128
EPAD = 327680
CHT = EPAD // (CW * NC * NS)
CROWS = EPAD // CW
WIN = 40
NBUF = 2
HIST = 10240
HSL = HIST // NS
NPAD = 10240
RSL = NPAD // NS
DPAD = 10016

_mesh = plsc.VectorSubcoreMesh(
    core_axis_name="c", subcore_axis_name="s", num_cores=NC, num_subcores=NS
)


@functools.partial(
    pl.kernel,
    out_type=jax.ShapeDtypeStruct((2, HIST), jnp.float32),
    mesh=_mesh,
    scratch_types=[
        pltpu.VMEM((ROWS_A, CWA), jnp.int32),
        pltpu.VMEM((CWA,), jnp.float32),
        pltpu.VMEM_SHARED((HIST,), jnp.float32),
    ],
)
def _deg_kernel(e3, ones_h, zeros_h, out, idx_v, ones_v, hist_sh):
    c = lax.axis_index("c")
    s = lax.axis_index("s")
    pltpu.sync_copy(ones_h, ones_v)
    pltpu.sync_copy(zeros_h, hist_sh.at[pl.ds(s * HSL, HSL)])
    pltpu.sync_copy(e3.at[c, pl.ds(s * ROWS_A, ROWS_A), :], idx_v)
    plsc.subcore_barrier()

    def body(j, carry):
        pltpu.sync_copy(ones_v, hist_sh.at[idx_v.at[j]], add=True)
        return carry

    lax.fori_loop(0, ROWS_A, body, 0)
    plsc.subcore_barrier()
    pltpu.sync_copy(hist_sh.at[pl.ds(s * HSL, HSL)], out.at[c, pl.ds(s * HSL, HSL)])


@functools.partial(
    pl.kernel,
    out_type=jax.ShapeDtypeStruct((NC, NPAD, D), jnp.float32),
    mesh=_mesh,
    scratch_types=[
        pltpu.VMEM((WIN, CW), jnp.int32),
        pltpu.VMEM((WIN, CW), jnp.int32),
        [pltpu.VMEM((CW, D), jnp.float32)] * NBUF,
        [pltpu.SemaphoreType.DMA] * NBUF,
        [pltpu.SemaphoreType.DMA] * NBUF,
        pltpu.VMEM_SHARED((NPAD, D), jnp.float32),
    ],
)
def _agg_kernel(feat, e3, zrows, out, sidx_v, didx_v, rows, gsems, ssems, agg_sh):
    c = lax.axis_index("c")
    s = lax.axis_index("s")
    rbase = (c * NS + s) * CHT
    pltpu.sync_copy(zrows, agg_sh.at[pl.ds(s * RSL, RSL), :])
    plsc.subcore_barrier()

    for h in range(CHT // WIN):
        pltpu.sync_copy(e3.at[0, pl.ds(rbase + h * WIN, WIN), :], sidx_v)
        pltpu.sync_copy(e3.at[1, pl.ds(rbase + h * WIN, WIN), :], didx_v)
        for b in range(NBUF):
            pltpu.async_copy(feat.at[sidx_v.at[b]], rows[b], gsems[b])

        def body(jj, carry):
            j0 = jj * NBUF
            for b in range(NBUF):
                pltpu.make_async_copy(
                    feat.at[sidx_v.at[j0 + b]], rows[b], gsems[b]).wait()
                pltpu.async_copy(
                    rows[b], agg_sh.at[didx_v.at[j0 + b]], ssems[b], add=True)
            for b in range(NBUF):

                @pl.when(j0 + NBUF + b < WIN)
                def _():
                    pltpu.make_async_copy(
                        rows[b], agg_sh.at[didx_v.at[j0 + b]], ssems[b]).wait()
                    pltpu.async_copy(
                        feat.at[sidx_v.at[j0 + NBUF + b]], rows[b], gsems[b])

            return carry

        lax.fori_loop(0, WIN // NBUF, body, 0)
        for b in range(NBUF):
            pltpu.make_async_copy(
                rows[b], agg_sh.at[didx_v.at[WIN - NBUF + b]], ssems[b]).wait()

    plsc.subcore_barrier()
    pltpu.sync_copy(
        agg_sh.at[pl.ds(s * RSL, RSL), :], out.at[c, pl.ds(s * RSL, RSL), :]
    )


def _mm(a, b):
    return lax.dot_general(
        a, b, (((1,), (0,)), ((), ())),
        preferred_element_type=jnp.float32,
    )


BR = 1000


def _pre_body(x_ref, w_ref, b_ref, o_ref):
    o_ref[...] = _mm(x_ref[...], w_ref[...]) + b_ref[...]


_pre_call = pl.pallas_call(
    _pre_body,
    grid=(N // BR,),
    in_specs=[
        pl.BlockSpec((BR, D), lambda i: (i, 0)),
        pl.BlockSpec((D, D), lambda i: (0, 0)),
        pl.BlockSpec((1, D), lambda i: (0, 0)),
    ],
    out_specs=pl.BlockSpec((BR, D), lambda i: (i, 0)),
    out_shape=jax.ShapeDtypeStruct((N, D), jnp.float32),
)


def _scale_body(f_ref, deg_ref, o_ref):
    norm = lax.rsqrt(jnp.maximum(deg_ref[...], 1.0))
    o_ref[...] = f_ref[...] * norm


_scale_call = pl.pallas_call(
    _scale_body,
    grid=(N // BR,),
    in_specs=[
        pl.BlockSpec((BR, D), lambda i: (i, 0)),
        pl.BlockSpec((BR, 1), lambda i: (i, 0)),
    ],
    out_specs=pl.BlockSpec((BR, D), lambda i: (i, 0)),
    out_shape=jax.ShapeDtypeStruct((N, D), jnp.float32),
)


def _post_body(aggp_ref, deg_ref, w_ref, xd_ref, w2_ref, b2_ref, wr_ref,
               br_ref, bias_ref, o_ref):
    a = aggp_ref[0] + aggp_ref[1]
    norm = lax.rsqrt(jnp.maximum(deg_ref[...], 1.0))
    rst = _mm(a, w_ref[...]) * norm
    res = _mm(_mm(xd_ref[...], w2_ref[...]) + b2_ref[...], wr_ref[...]) + br_ref[...]
    o_ref[...] = rst + res + bias_ref[...]


_post_call = pl.pallas_call(
    _post_body,
    grid=(N // BR,),
    in_specs=[
        pl.BlockSpec((NC, BR, D), lambda i: (0, i, 0)),
        pl.BlockSpec((BR, 1), lambda i: (i, 0)),
        pl.BlockSpec((D, D), lambda i: (0, 0)),
        pl.BlockSpec((BR, D), lambda i: (i, 0)),
        pl.BlockSpec((D, D), lambda i: (0, 0)),
        pl.BlockSpec((1, D), lambda i: (0, 0)),
        pl.BlockSpec((D, D), lambda i: (0, 0)),
        pl.BlockSpec((1, D), lambda i: (0, 0)),
        pl.BlockSpec((1, D), lambda i: (0, 0)),
    ],
    out_specs=pl.BlockSpec((BR, D), lambda i: (i, 0)),
    out_shape=jax.ShapeDtypeStruct((N, D), jnp.float32),
)


def kernel(x_src, x_dst, edge_index, W1, b1, W2, b2, W_res, b_res, weight, bias):
    ei = edge_index.astype(jnp.int32)
    e3d = ei.reshape(2, ROWS_ALL_A, CWA)
    pad = jnp.stack([
        jnp.arange(EPAD - E, dtype=jnp.int32) % N,
        DPAD + (jnp.arange(EPAD - E, dtype=jnp.int32) % (NPAD - DPAD)),
    ])
    e3 = jnp.concatenate([ei, pad], axis=1).reshape(2, CROWS, CW)
    ones_h = jnp.ones((CWA,), jnp.float32)
    zhist = jnp.zeros((HSL,), jnp.float32)
    zrows = jnp.zeros((RSL, D), jnp.float32)

    deg = _deg_kernel(e3d, ones_h, zhist)
    feat_raw = _pre_call(x_src, W1, b1.reshape(1, D))
    feat = _scale_call(feat_raw, deg[0, :N, None])
    aggp = _agg_kernel(feat, e3, zrows)[:, :N, :]
    out = _post_call(aggp, deg[1, :N, None], weight, x_dst, W2,
                     b2.reshape(1, D), W_res, b_res.reshape(1, D),
                     bias.reshape(1, D))
    return out


# speedup vs baseline: 1.1886x; 1.1886x over previous
import functools

import jax
import jax.numpy as jnp
from jax import lax
from jax.experimental import pallas as pl
from jax.experimental.pallas import tpu as pltpu
from jax.experimental.pallas import tpu_sc as plsc

N = 10000
E = 320000
D = 128
NC = 2
NS = 16
CWA = 125
ROWS_ALL_A = E // CWA
ROWS_A = ROWS_ALL_A // NS
CW = 128
EPAD = 327680
CHT = EPAD // (CW * NC * NS)
CROWS = EPAD // CW
WIN = 40
NBUF = 2
HIST = 10240
HSL = HIST // NS
NPAD = 10240
RSL = NPAD // NS
DPAD = 10016

_mesh = plsc.VectorSubcoreMesh(
    core_axis_name="c", subcore_axis_name="s", num_cores=NC, num_subcores=NS
)


@functools.partial(
    pl.kernel,
    out_type=jax.ShapeDtypeStruct((2, HIST), jnp.float32),
    mesh=_mesh,
    scratch_types=[
        pltpu.VMEM((ROWS_A, CWA), jnp.int32),
        pltpu.VMEM((CWA,), jnp.float32),
        pltpu.VMEM_SHARED((HIST,), jnp.float32),
    ],
)
def _deg_kernel(e3, ones_h, zeros_h, out, idx_v, ones_v, hist_sh):
    c = lax.axis_index("c")
    s = lax.axis_index("s")
    pltpu.sync_copy(ones_h, ones_v)
    pltpu.sync_copy(zeros_h, hist_sh.at[pl.ds(s * HSL, HSL)])
    pltpu.sync_copy(e3.at[c, pl.ds(s * ROWS_A, ROWS_A), :], idx_v)
    plsc.subcore_barrier()

    def body(j, carry):
        pltpu.sync_copy(ones_v, hist_sh.at[idx_v.at[j]], add=True)
        return carry

    lax.fori_loop(0, ROWS_A, body, 0)
    plsc.subcore_barrier()
    pltpu.sync_copy(hist_sh.at[pl.ds(s * HSL, HSL)], out.at[c, pl.ds(s * HSL, HSL)])


@functools.partial(
    pl.kernel,
    out_type=jax.ShapeDtypeStruct((NC, NPAD, D), jnp.float32),
    mesh=_mesh,
    scratch_types=[
        pltpu.VMEM((WIN, CW), jnp.int32),
        pltpu.VMEM((WIN, CW), jnp.int32),
        [pltpu.VMEM((CW, D), jnp.float32)] * NBUF,
        [pltpu.SemaphoreType.DMA] * NBUF,
        [pltpu.SemaphoreType.DMA] * NBUF,
        pltpu.VMEM_SHARED((NPAD, D), jnp.float32),
    ],
)
def _agg_kernel(feat, e3, zrows, out, sidx_v, didx_v, rows, gsems, ssems, agg_sh):
    c = lax.axis_index("c")
    s = lax.axis_index("s")
    rbase = (c * NS + s) * CHT
    pltpu.sync_copy(zrows, agg_sh.at[pl.ds(s * RSL, RSL), :])
    plsc.subcore_barrier()

    for h in range(CHT // WIN):
        pltpu.sync_copy(e3.at[0, pl.ds(rbase + h * WIN, WIN), :], sidx_v)
        pltpu.sync_copy(e3.at[1, pl.ds(rbase + h * WIN, WIN), :], didx_v)
        for b in range(NBUF):
            pltpu.async_copy(feat.at[sidx_v.at[b]], rows[b], gsems[b])

        def body(jj, carry):
            j0 = jj * NBUF
            for b in range(NBUF):
                pltpu.make_async_copy(
                    feat.at[sidx_v.at[j0 + b]], rows[b], gsems[b]).wait()
                pltpu.sync_copy(rows[b], agg_sh.at[didx_v.at[j0 + b]], add=True)

                @pl.when(j0 + NBUF + b < WIN)
                def _():
                    pltpu.async_copy(
                        feat.at[sidx_v.at[j0 + NBUF + b]], rows[b], gsems[b])

            return carry

        lax.fori_loop(0, WIN // NBUF, body, 0)

    plsc.subcore_barrier()
    pltpu.sync_copy(
        agg_sh.at[pl.ds(s * RSL, RSL), :], out.at[c, pl.ds(s * RSL, RSL), :]
    )


def _mm(a, b):
    return lax.dot_general(
        a, b, (((1,), (0,)), ((), ())),
        preferred_element_type=jnp.float32,
    )


BR = 1000


def _pre_body(x_ref, w_ref, b_ref, o_ref):
    o_ref[...] = _mm(x_ref[...], w_ref[...]) + b_ref[...]


_pre_call = pl.pallas_call(
    _pre_body,
    grid=(N // BR,),
    in_specs=[
        pl.BlockSpec((BR, D), lambda i: (i, 0)),
        pl.BlockSpec((D, D), lambda i: (0, 0)),
        pl.BlockSpec((1, D), lambda i: (0, 0)),
    ],
    out_specs=pl.BlockSpec((BR, D), lambda i: (i, 0)),
    out_shape=jax.ShapeDtypeStruct((N, D), jnp.float32),
)


def _scale_body(f_ref, deg_ref, o_ref):
    norm = lax.rsqrt(jnp.maximum(deg_ref[...], 1.0))
    o_ref[...] = f_ref[...] * norm


_scale_call = pl.pallas_call(
    _scale_body,
    grid=(N // BR,),
    in_specs=[
        pl.BlockSpec((BR, D), lambda i: (i, 0)),
        pl.BlockSpec((BR, 1), lambda i: (i, 0)),
    ],
    out_specs=pl.BlockSpec((BR, D), lambda i: (i, 0)),
    out_shape=jax.ShapeDtypeStruct((N, D), jnp.float32),
)


def _post_body(aggp_ref, deg_ref, w_ref, xd_ref, w2_ref, b2_ref, wr_ref,
               br_ref, bias_ref, o_ref):
    a = aggp_ref[0] + aggp_ref[1]
    norm = lax.rsqrt(jnp.maximum(deg_ref[...], 1.0))
    rst = _mm(a, w_ref[...]) * norm
    res = _mm(_mm(xd_ref[...], w2_ref[...]) + b2_ref[...], wr_ref[...]) + br_ref[...]
    o_ref[...] = rst + res + bias_ref[...]


_post_call = pl.pallas_call(
    _post_body,
    grid=(N // BR,),
    in_specs=[
        pl.BlockSpec((NC, BR, D), lambda i: (0, i, 0)),
        pl.BlockSpec((BR, 1), lambda i: (i, 0)),
        pl.BlockSpec((D, D), lambda i: (0, 0)),
        pl.BlockSpec((BR, D), lambda i: (i, 0)),
        pl.BlockSpec((D, D), lambda i: (0, 0)),
        pl.BlockSpec((1, D), lambda i: (0, 0)),
        pl.BlockSpec((D, D), lambda i: (0, 0)),
        pl.BlockSpec((1, D), lambda i: (0, 0)),
        pl.BlockSpec((1, D), lambda i: (0, 0)),
    ],
    out_specs=pl.BlockSpec((BR, D), lambda i: (i, 0)),
    out_shape=jax.ShapeDtypeStruct((N, D), jnp.float32),
)


def kernel(x_src, x_dst, edge_index, W1, b1, W2, b2, W_res, b_res, weight, bias):
    ei = edge_index.astype(jnp.int32)
    e3d = ei.reshape(2, ROWS_ALL_A, CWA)
    pad = jnp.stack([
        jnp.arange(EPAD - E, dtype=jnp.int32) % N,
        DPAD + (jnp.arange(EPAD - E, dtype=jnp.int32) % (NPAD - DPAD)),
    ])
    e3 = jnp.concatenate([ei, pad], axis=1).reshape(2, CROWS, CW)
    ones_h = jnp.ones((CWA,), jnp.float32)
    zhist = jnp.zeros((HSL,), jnp.float32)
    zrows = jnp.zeros((RSL, D), jnp.float32)

    deg = _deg_kernel(e3d, ones_h, zhist)
    feat_raw = _pre_call(x_src, W1, b1.reshape(1, D))
    feat = _scale_call(feat_raw, deg[0, :N, None])
    aggp = _agg_kernel(feat, e3, zrows)[:, :N, :]
    out = _post_call(aggp, deg[1, :N, None], weight, x_dst, W2,
                     b2.reshape(1, D), W_res, b_res.reshape(1, D),
                     bias.reshape(1, D))
    return out


# speedup vs baseline: 1.2280x; 1.0331x over previous
import functools

import jax
import jax.numpy as jnp
from jax import lax
from jax.experimental import pallas as pl
from jax.experimental.pallas import tpu as pltpu
from jax.experimental.pallas import tpu_sc as plsc

N = 10000
E = 320000
D = 128
NC = 2
NS = 16
CW = 128
EPAD = 327680
CHT = EPAD // (CW * NC * NS)
CROWS = EPAD // CW
ROWS_A = CROWS // NS
WIN = 40
NBUF = 2
HIST = 10240
HSL = HIST // NS
NPAD = 10240
RSL = NPAD // NS

_mesh = plsc.VectorSubcoreMesh(
    core_axis_name="c", subcore_axis_name="s", num_cores=NC, num_subcores=NS
)


@functools.partial(
    pl.kernel,
    out_type=jax.ShapeDtypeStruct((2, HIST), jnp.float32),
    mesh=_mesh,
    scratch_types=[
        pltpu.VMEM((ROWS_A, CW), jnp.int32),
        pltpu.VMEM((CW,), jnp.float32),
        pltpu.VMEM_SHARED((HIST,), jnp.float32),
    ],
)
def _deg_kernel(e3, ones_h, zeros_h, out, idx_v, ones_v, hist_sh):
    c = lax.axis_index("c")
    s = lax.axis_index("s")
    pltpu.sync_copy(ones_h, ones_v)
    pltpu.sync_copy(zeros_h, hist_sh.at[pl.ds(s * HSL, HSL)])
    pltpu.sync_copy(e3.at[c, pl.ds(s * ROWS_A, ROWS_A), :], idx_v)
    plsc.subcore_barrier()

    def body(j, carry):
        pltpu.sync_copy(ones_v, hist_sh.at[idx_v.at[j]], add=True)
        return carry

    lax.fori_loop(0, ROWS_A, body, 0)
    plsc.subcore_barrier()
    pltpu.sync_copy(hist_sh.at[pl.ds(s * HSL, HSL)], out.at[c, pl.ds(s * HSL, HSL)])


@functools.partial(
    pl.kernel,
    out_type=jax.ShapeDtypeStruct((NC, NPAD, D), jnp.float32),
    mesh=_mesh,
    scratch_types=[
        pltpu.VMEM((WIN, CW), jnp.int32),
        pltpu.VMEM((WIN, CW), jnp.int32),
        [pltpu.VMEM((CW, D), jnp.float32)] * NBUF,
        [pltpu.SemaphoreType.DMA] * NBUF,
        [pltpu.SemaphoreType.DMA] * NBUF,
        pltpu.VMEM_SHARED((NPAD, D), jnp.float32),
    ],
)
def _agg_kernel(feat, e3, zrows, out, sidx_v, didx_v, rows, gsems, ssems, agg_sh):
    c = lax.axis_index("c")
    s = lax.axis_index("s")
    rbase = (c * NS + s) * CHT
    pltpu.sync_copy(zrows, agg_sh.at[pl.ds(s * RSL, RSL), :])
    plsc.subcore_barrier()

    for h in range(CHT // WIN):
        pltpu.sync_copy(e3.at[0, pl.ds(rbase + h * WIN, WIN), :], sidx_v)
        pltpu.sync_copy(e3.at[1, pl.ds(rbase + h * WIN, WIN), :], didx_v)
        for b in range(NBUF):
            pltpu.async_copy(feat.at[sidx_v.at[b]], rows[b], gsems[b])

        def body(jj, carry):
            j0 = jj * NBUF
            for b in range(NBUF):
                pltpu.make_async_copy(
                    feat.at[sidx_v.at[j0 + b]], rows[b], gsems[b]).wait()
                pltpu.sync_copy(rows[b], agg_sh.at[didx_v.at[j0 + b]], add=True)

                @pl.when(j0 + NBUF + b < WIN)
                def _():
                    pltpu.async_copy(
                        feat.at[sidx_v.at[j0 + NBUF + b]], rows[b], gsems[b])

            return carry

        lax.fori_loop(0, WIN // NBUF, body, 0)

    plsc.subcore_barrier()
    pltpu.sync_copy(
        agg_sh.at[pl.ds(s * RSL, RSL), :], out.at[c, pl.ds(s * RSL, RSL), :]
    )


def _mm(a, b):
    return lax.dot_general(
        a, b, (((1,), (0,)), ((), ())),
        preferred_element_type=jnp.float32,
    )


BR = 1000


def _pre_body(x_ref, w_ref, b_ref, o_ref):
    o_ref[...] = _mm(x_ref[...], w_ref[...]) + b_ref[...]


_pre_call = pl.pallas_call(
    _pre_body,
    grid=(N // BR,),
    in_specs=[
        pl.BlockSpec((BR, D), lambda i: (i, 0)),
        pl.BlockSpec((D, D), lambda i: (0, 0)),
        pl.BlockSpec((1, D), lambda i: (0, 0)),
    ],
    out_specs=pl.BlockSpec((BR, D), lambda i: (i, 0)),
    out_shape=jax.ShapeDtypeStruct((N, D), jnp.float32),
)


def _scale_body(f_ref, deg_ref, o_ref):
    norm = lax.rsqrt(jnp.maximum(deg_ref[...], 1.0))
    o_ref[...] = f_ref[...] * norm


_scale_call = pl.pallas_call(
    _scale_body,
    grid=(N // BR,),
    in_specs=[
        pl.BlockSpec((BR, D), lambda i: (i, 0)),
        pl.BlockSpec((BR, 1), lambda i: (i, 0)),
    ],
    out_specs=pl.BlockSpec((BR, D), lambda i: (i, 0)),
    out_shape=jax.ShapeDtypeStruct((NPAD, D), jnp.float32),
)


def _post_body(aggp_ref, deg_ref, w_ref, xd_ref, w2_ref, b2_ref, wr_ref,
               br_ref, bias_ref, o_ref):
    a = aggp_ref[0] + aggp_ref[1]
    norm = lax.rsqrt(jnp.maximum(deg_ref[...], 1.0))
    rst = _mm(a, w_ref[...]) * norm
    res = _mm(_mm(xd_ref[...], w2_ref[...]) + b2_ref[...], wr_ref[...]) + br_ref[...]
    o_ref[...] = rst + res + bias_ref[...]


_post_call = pl.pallas_call(
    _post_body,
    grid=(N // BR,),
    in_specs=[
        pl.BlockSpec((NC, BR, D), lambda i: (0, i, 0)),
        pl.BlockSpec((BR, 1), lambda i: (i, 0)),
        pl.BlockSpec((D, D), lambda i: (0, 0)),
        pl.BlockSpec((BR, D), lambda i: (i, 0)),
        pl.BlockSpec((D, D), lambda i: (0, 0)),
        pl.BlockSpec((1, D), lambda i: (0, 0)),
        pl.BlockSpec((D, D), lambda i: (0, 0)),
        pl.BlockSpec((1, D), lambda i: (0, 0)),
        pl.BlockSpec((1, D), lambda i: (0, 0)),
    ],
    out_specs=pl.BlockSpec((BR, D), lambda i: (i, 0)),
    out_shape=jax.ShapeDtypeStruct((N, D), jnp.float32),
)


def kernel(x_src, x_dst, edge_index, W1, b1, W2, b2, W_res, b_res, weight, bias):
    ei = edge_index.astype(jnp.int32)
    spread = N + (jnp.arange(EPAD - E, dtype=jnp.int32) % (NPAD - N))
    pad = jnp.stack([spread, spread])
    e3 = jnp.concatenate([ei, pad], axis=1).reshape(2, CROWS, CW)
    ones_h = jnp.ones((CW,), jnp.float32)
    zhist = jnp.zeros((HSL,), jnp.float32)
    zrows = jnp.zeros((RSL, D), jnp.float32)

    deg = _deg_kernel(e3, ones_h, zhist)
    feat_raw = _pre_call(x_src, W1, b1.reshape(1, D))
    feat = _scale_call(feat_raw, deg[0, :N, None])
    aggp = _agg_kernel(feat, e3, zrows)
    out = _post_call(aggp, deg[1, :N, None], weight, x_dst, W2,
                     b2.reshape(1, D), W_res, b_res.reshape(1, D),
                     bias.reshape(1, D))
    return out


# speedup vs baseline: 1.2397x; 1.0095x over previous
import functools

import jax
import jax.numpy as jnp
from jax import lax
from jax.experimental import pallas as pl
from jax.experimental.pallas import tpu as pltpu
from jax.experimental.pallas import tpu_sc as plsc

N = 10000
E = 320000
D = 128
NC = 2
NS = 16
CW = 128
EPAD = 327680
CHT = EPAD // (CW * NC * NS)
CROWS = EPAD // CW
ROWS_A = CROWS // NS
WIN = 40
NBUF = 2
HIST = 10240
HSL = HIST // NS
NPAD = 10240
RSL = NPAD // NS

_mesh = plsc.VectorSubcoreMesh(
    core_axis_name="c", subcore_axis_name="s", num_cores=NC, num_subcores=NS
)


@functools.partial(
    pl.kernel,
    out_type=jax.ShapeDtypeStruct((2, HIST), jnp.float32),
    mesh=_mesh,
    scratch_types=[
        pltpu.VMEM((ROWS_A, CW), jnp.int32),
        pltpu.VMEM((CW,), jnp.float32),
        pltpu.VMEM_SHARED((HIST,), jnp.float32),
    ],
)
def _deg_kernel(e3, ones_h, zeros_h, out, idx_v, ones_v, hist_sh):
    c = lax.axis_index("c")
    s = lax.axis_index("s")
    pltpu.sync_copy(ones_h, ones_v)
    pltpu.sync_copy(zeros_h, hist_sh.at[pl.ds(s * HSL, HSL)])
    pltpu.sync_copy(e3.at[c, pl.ds(s * ROWS_A, ROWS_A), :], idx_v)
    plsc.subcore_barrier()

    def body(j, carry):
        pltpu.sync_copy(ones_v, hist_sh.at[idx_v.at[j]], add=True)
        return carry

    lax.fori_loop(0, ROWS_A, body, 0)
    plsc.subcore_barrier()
    pltpu.sync_copy(hist_sh.at[pl.ds(s * HSL, HSL)], out.at[c, pl.ds(s * HSL, HSL)])


@functools.partial(
    pl.kernel,
    out_type=jax.ShapeDtypeStruct((NC, NPAD, D), jnp.float32),
    mesh=_mesh,
    scratch_types=[
        pltpu.VMEM((WIN, CW), jnp.int32),
        pltpu.VMEM((WIN, CW), jnp.int32),
        [pltpu.VMEM((CW, D), jnp.float32)] * NBUF,
        [pltpu.SemaphoreType.DMA] * NBUF,
        [pltpu.SemaphoreType.DMA] * NBUF,
        pltpu.VMEM_SHARED((NPAD, D), jnp.float32),
    ],
)
def _agg_kernel(feat, e3, zrows, out, sidx_v, didx_v, rows, gsems, ssems, agg_sh):
    c = lax.axis_index("c")
    s = lax.axis_index("s")
    rbase = (c * NS + s) * CHT
    pltpu.sync_copy(zrows, agg_sh.at[pl.ds(s * RSL, RSL), :])
    plsc.subcore_barrier()

    for h in range(CHT // WIN):
        pltpu.sync_copy(e3.at[0, pl.ds(rbase + h * WIN, WIN), :], sidx_v)
        pltpu.sync_copy(e3.at[1, pl.ds(rbase + h * WIN, WIN), :], didx_v)
        for b in range(NBUF):
            pltpu.async_copy(feat.at[sidx_v.at[b]], rows[b], gsems[b])

        def body(jj, carry):
            j0 = jj * NBUF
            for b in range(NBUF):
                pltpu.make_async_copy(
                    feat.at[sidx_v.at[j0 + b]], rows[b], gsems[b]).wait()
                pltpu.sync_copy(rows[b], agg_sh.at[didx_v.at[j0 + b]], add=True)

                @pl.when(j0 + NBUF + b < WIN)
                def _():
                    pltpu.async_copy(
                        feat.at[sidx_v.at[j0 + NBUF + b]], rows[b], gsems[b])

            return carry

        lax.fori_loop(0, WIN // NBUF, body, 0)

    plsc.subcore_barrier()
    pltpu.sync_copy(
        agg_sh.at[pl.ds(s * RSL, RSL), :], out.at[c, pl.ds(s * RSL, RSL), :]
    )


def _mm(a, b):
    return lax.dot_general(
        a, b, (((1,), (0,)), ((), ())),
        preferred_element_type=jnp.float32,
    )


BR = 1000


def _pre_body(x_ref, w_ref, b_ref, deg_ref, o_ref):
    norm = lax.rsqrt(jnp.maximum(deg_ref[...], 1.0))
    o_ref[...] = (_mm(x_ref[...], w_ref[...]) + b_ref[...]) * norm


_pre_call = pl.pallas_call(
    _pre_body,
    grid=(N // BR,),
    in_specs=[
        pl.BlockSpec((BR, D), lambda i: (i, 0)),
        pl.BlockSpec((D, D), lambda i: (0, 0)),
        pl.BlockSpec((1, D), lambda i: (0, 0)),
        pl.BlockSpec((BR, 1), lambda i: (i, 0)),
    ],
    out_specs=pl.BlockSpec((BR, D), lambda i: (i, 0)),
    out_shape=jax.ShapeDtypeStruct((NPAD, D), jnp.float32),
)


def _post_body(aggp_ref, deg_ref, w_ref, xd_ref, w2_ref, b2_ref, wr_ref,
               br_ref, bias_ref, o_ref):
    a = aggp_ref[0] + aggp_ref[1]
    norm = lax.rsqrt(jnp.maximum(deg_ref[...], 1.0))
    rst = _mm(a, w_ref[...]) * norm
    res = _mm(_mm(xd_ref[...], w2_ref[...]) + b2_ref[...], wr_ref[...]) + br_ref[...]
    o_ref[...] = rst + res + bias_ref[...]


_post_call = pl.pallas_call(
    _post_body,
    grid=(N // BR,),
    in_specs=[
        pl.BlockSpec((NC, BR, D), lambda i: (0, i, 0)),
        pl.BlockSpec((BR, 1), lambda i: (i, 0)),
        pl.BlockSpec((D, D), lambda i: (0, 0)),
        pl.BlockSpec((BR, D), lambda i: (i, 0)),
        pl.BlockSpec((D, D), lambda i: (0, 0)),
        pl.BlockSpec((1, D), lambda i: (0, 0)),
        pl.BlockSpec((D, D), lambda i: (0, 0)),
        pl.BlockSpec((1, D), lambda i: (0, 0)),
        pl.BlockSpec((1, D), lambda i: (0, 0)),
    ],
    out_specs=pl.BlockSpec((BR, D), lambda i: (i, 0)),
    out_shape=jax.ShapeDtypeStruct((N, D), jnp.float32),
)


def kernel(x_src, x_dst, edge_index, W1, b1, W2, b2, W_res, b_res, weight, bias):
    ei = edge_index.astype(jnp.int32)
    spread = N + (jnp.arange(EPAD - E, dtype=jnp.int32) % (NPAD - N))
    pad = jnp.stack([spread, spread])
    e3 = jnp.concatenate([ei, pad], axis=1).reshape(2, CROWS, CW)
    ones_h = jnp.ones((CW,), jnp.float32)
    zhist = jnp.zeros((HSL,), jnp.float32)
    zrows = jnp.zeros((RSL, D), jnp.float32)

    deg = _deg_kernel(e3, ones_h, zhist)
    feat = _pre_call(x_src, W1, b1.reshape(1, D), deg[0, :N, None])
    aggp = _agg_kernel(feat, e3, zrows)
    out = _post_call(aggp, deg[1, :N, None], weight, x_dst, W2,
                     b2.reshape(1, D), W_res, b_res.reshape(1, D),
                     bias.reshape(1, D))
    return out


# speedup vs baseline: 1.2422x; 1.0020x over previous
import functools

import jax
import jax.numpy as jnp
from jax import lax
from jax.experimental import pallas as pl
from jax.experimental.pallas import tpu as pltpu
from jax.experimental.pallas import tpu_sc as plsc

N = 10000
E = 320000
D = 128
NC = 2
NS = 16
CW = 128
EPAD = 327680
CHT = EPAD // (CW * NC * NS)
CROWS = EPAD // CW
ROWS_A = CROWS // NS
WIN = 40
NBUF = 2
HIST = 10240
HSL = HIST // NS
NPAD = 10240
RSL = NPAD // NS

_mesh = plsc.VectorSubcoreMesh(
    core_axis_name="c", subcore_axis_name="s", num_cores=NC, num_subcores=NS
)


@functools.partial(
    pl.kernel,
    out_type=jax.ShapeDtypeStruct((2, HIST), jnp.float32),
    mesh=_mesh,
    scratch_types=[
        pltpu.VMEM((ROWS_A, CW), jnp.int32),
        pltpu.VMEM((CW,), jnp.float32),
        pltpu.VMEM_SHARED((HIST,), jnp.float32),
    ],
)
def _deg_kernel(e3, ones_h, zeros_h, out, idx_v, ones_v, hist_sh):
    c = lax.axis_index("c")
    s = lax.axis_index("s")
    pltpu.sync_copy(ones_h, ones_v)
    pltpu.sync_copy(zeros_h, hist_sh.at[pl.ds(s * HSL, HSL)])
    pltpu.sync_copy(e3.at[c, pl.ds(s * ROWS_A, ROWS_A), :], idx_v)
    plsc.subcore_barrier()

    def body(j, carry):
        pltpu.sync_copy(ones_v, hist_sh.at[idx_v.at[j]], add=True)
        return carry

    lax.fori_loop(0, ROWS_A, body, 0)
    plsc.subcore_barrier()
    pltpu.sync_copy(hist_sh.at[pl.ds(s * HSL, HSL)], out.at[c, pl.ds(s * HSL, HSL)])


@functools.partial(
    pl.kernel,
    out_type=jax.ShapeDtypeStruct((NC, NPAD, D), jnp.float32),
    mesh=_mesh,
    scratch_types=[
        pltpu.VMEM((WIN, CW), jnp.int32),
        pltpu.VMEM((WIN, CW), jnp.int32),
        [pltpu.VMEM((CW, D), jnp.float32)] * NBUF,
        [pltpu.SemaphoreType.DMA] * NBUF,
        [pltpu.SemaphoreType.DMA] * NBUF,
        pltpu.VMEM_SHARED((NPAD, D), jnp.float32),
    ],
)
def _agg_kernel(feat, e3, zrows, out, sidx_v, didx_v, rows, gsems, ssems, agg_sh):
    c = lax.axis_index("c")
    s = lax.axis_index("s")
    rbase = (c * NS + s) * CHT
    pltpu.sync_copy(zrows, agg_sh.at[pl.ds(s * RSL, RSL), :])
    plsc.subcore_barrier()

    for h in range(CHT // WIN):
        pltpu.sync_copy(e3.at[0, pl.ds(rbase + h * WIN, WIN), :], sidx_v)
        pltpu.sync_copy(e3.at[1, pl.ds(rbase + h * WIN, WIN), :], didx_v)
        for b in range(NBUF):
            pltpu.async_copy(feat.at[sidx_v.at[b]], rows[b], gsems[b])

        def body(jj, carry):
            j0 = jj * NBUF
            for b in range(NBUF):
                pltpu.make_async_copy(
                    feat.at[sidx_v.at[j0 + b]], rows[b], gsems[b]).wait()
                pltpu.sync_copy(rows[b], agg_sh.at[didx_v.at[j0 + b]], add=True)

                @pl.when(j0 + NBUF + b < WIN)
                def _():
                    pltpu.async_copy(
                        feat.at[sidx_v.at[j0 + NBUF + b]], rows[b], gsems[b])

            return carry

        lax.fori_loop(0, WIN // NBUF, body, 0)

    plsc.subcore_barrier()
    pltpu.sync_copy(
        agg_sh.at[pl.ds(s * RSL, RSL), :], out.at[c, pl.ds(s * RSL, RSL), :]
    )


def _mm(a, b):
    return lax.dot_general(
        a.astype(jnp.bfloat16), b.astype(jnp.bfloat16), (((1,), (0,)), ((), ())),
        preferred_element_type=jnp.float32,
    )


BR = 1000


def _pre_body(x_ref, w_ref, b_ref, deg_ref, o_ref):
    norm = lax.rsqrt(jnp.maximum(deg_ref[...], 1.0))
    o_ref[...] = (_mm(x_ref[...], w_ref[...]) + b_ref[...]) * norm


_pre_call = pl.pallas_call(
    _pre_body,
    grid=(N // BR,),
    in_specs=[
        pl.BlockSpec((BR, D), lambda i: (i, 0)),
        pl.BlockSpec((D, D), lambda i: (0, 0)),
        pl.BlockSpec((1, D), lambda i: (0, 0)),
        pl.BlockSpec((BR, 1), lambda i: (i, 0)),
    ],
    out_specs=pl.BlockSpec((BR, D), lambda i: (i, 0)),
    out_shape=jax.ShapeDtypeStruct((NPAD, D), jnp.float32),
)


def _post_body(aggp_ref, deg_ref, w_ref, xd_ref, w2_ref, b2_ref, wr_ref,
               br_ref, bias_ref, o_ref, w2r_s, c_s):
    i = pl.program_id(0)

    @pl.when(i == 0)
    def _():
        w2r_s[...] = _mm(w2_ref[...], wr_ref[...])
        c_s[...] = _mm(b2_ref[...], wr_ref[...]) + br_ref[...] + bias_ref[...]

    a = aggp_ref[0] + aggp_ref[1]
    norm = lax.rsqrt(jnp.maximum(deg_ref[...], 1.0))
    rst = _mm(a, w_ref[...]) * norm
    o_ref[...] = rst + _mm(xd_ref[...], w2r_s[...]) + c_s[...]


_post_call = pl.pallas_call(
    _post_body,
    grid=(N // BR,),
    in_specs=[
        pl.BlockSpec((NC, BR, D), lambda i: (0, i, 0)),
        pl.BlockSpec((BR, 1), lambda i: (i, 0)),
        pl.BlockSpec((D, D), lambda i: (0, 0)),
        pl.BlockSpec((BR, D), lambda i: (i, 0)),
        pl.BlockSpec((D, D), lambda i: (0, 0)),
        pl.BlockSpec((1, D), lambda i: (0, 0)),
        pl.BlockSpec((D, D), lambda i: (0, 0)),
        pl.BlockSpec((1, D), lambda i: (0, 0)),
        pl.BlockSpec((1, D), lambda i: (0, 0)),
    ],
    out_specs=pl.BlockSpec((BR, D), lambda i: (i, 0)),
    out_shape=jax.ShapeDtypeStruct((N, D), jnp.float32),
    scratch_shapes=[
        pltpu.VMEM((D, D), jnp.float32),
        pltpu.VMEM((1, D), jnp.float32),
    ],
)


def kernel(x_src, x_dst, edge_index, W1, b1, W2, b2, W_res, b_res, weight, bias):
    ei = edge_index.astype(jnp.int32)
    spread = N + (jnp.arange(EPAD - E, dtype=jnp.int32) % (NPAD - N))
    pad = jnp.stack([spread, spread])
    e3 = jnp.concatenate([ei, pad], axis=1).reshape(2, CROWS, CW)
    ones_h = jnp.ones((CW,), jnp.float32)
    zhist = jnp.zeros((HSL,), jnp.float32)
    zrows = jnp.zeros((RSL, D), jnp.float32)

    deg = _deg_kernel(e3, ones_h, zhist)
    feat = _pre_call(x_src, W1, b1.reshape(1, D), deg[0, :N, None])
    aggp = _agg_kernel(feat, e3, zrows)
    out = _post_call(aggp, deg[1, :N, None], weight, x_dst, W2,
                     b2.reshape(1, D), W_res, b_res.reshape(1, D),
                     bias.reshape(1, D))
    return out
